# baseline (speedup 1.0000x reference)
import functools

import jax
import jax.numpy as jnp
from jax import lax
from jax.experimental import pallas as pl
from jax.experimental.pallas import tpu as pltpu
from jax.experimental.pallas import tpu_sc as plsc

B = 16384
D = 64
L = 16
NC = 2
NS = 16
NW = NC * NS
BPW = B // NW
CH = 128
HALF = 256
NCH = HALF // CH
NG = HALF // L

NE = 1000000
NB = (NE + 127) // 128
NROW2 = NB * 64

_params = pltpu.CompilerParams(
    needs_layout_passes=False,
    use_tc_tiling_on_sc=True,
    disable_bounds_checks=True,
)
_mesh = plsc.VectorSubcoreMesh(core_axis_name="c", subcore_axis_name="s")


@functools.partial(
    pl.kernel,
    mesh=_mesh,
    compiler_params=_params,
    out_type=(
        jax.ShapeDtypeStruct((NROW2, 128), jnp.float32),
        jax.ShapeDtypeStruct((NROW2, 128), jnp.float32),
    ),
    scratch_types=[
        pltpu.VMEM((2, D, 128), jnp.float32),
        pltpu.VMEM((2, D, 128), jnp.float32),
        pltpu.SemaphoreType.DMA,
        pltpu.SemaphoreType.DMA,
    ],
)
def _repack_sc(entt_hbm, relt_hbm, e2_hbm, r2_hbm, tin_v, tout_v,
               sem_in, sem_out):
    wid = lax.axis_index("s") * NC + lax.axis_index("c")
    nk = (NB - wid + NW - 1) // NW

    lane = lax.iota(jnp.int32, L)

    for src_hbm, dst_hbm in ((entt_hbm, e2_hbm), (relt_hbm, r2_hbm)):

        def fire_in(k, slot, src_hbm=src_hbm):
            c = wid + k * NW
            pltpu.async_copy(
                src_hbm.at[:, pl.ds(c * 128, 128)], tin_v.at[slot], sem_in
            )

        fire_in(0, 0)

        def k_body(k, carry, src_hbm=src_hbm, dst_hbm=dst_hbm):
            slot = k & 1
            pltpu.make_async_copy(
                src_hbm.at[:, pl.ds(0, 128)], tin_v.at[slot], sem_in
            ).wait()

            @pl.when(k + 1 < nk)
            def _():
                fire_in(k + 1, 1 - slot)

            @pl.when(k >= 2)
            def _():
                pltpu.make_async_copy(
                    src_hbm.at[:, pl.ds(0, 128)], tout_v.at[slot], sem_out
                ).wait()

            def r_body(kp, _):
                e0 = jnp.full((L,), 2 * kp, jnp.int32)
                e1 = e0 + 1
                for j in range(D // L):
                    dvec = j * L + lane
                    tout_v[slot, kp, pl.ds(j * L, L)] = plsc.load_gather(
                        tin_v, [jnp.full((L,), slot, jnp.int32), dvec, e0]
                    )
                    tout_v[slot, kp, pl.ds(D + j * L, L)] = plsc.load_gather(
                        tin_v, [jnp.full((L,), slot, jnp.int32), dvec, e1]
                    )
                return 0

            lax.fori_loop(0, D, r_body, 0)

            c = wid + k * NW
            pltpu.async_copy(
                tout_v.at[slot], dst_hbm.at[pl.ds(c * 64, 64)], sem_out
            )
            return carry

        lax.fori_loop(0, nk, k_body, 0)

        @pl.when(nk >= 1)
        def _():
            pltpu.make_async_copy(
                src_hbm.at[:, pl.ds(0, 128)], tout_v.at[0], sem_out
            ).wait()

        @pl.when(nk >= 2)
        def _():
            pltpu.make_async_copy(
                src_hbm.at[:, pl.ds(0, 128)], tout_v.at[1], sem_out
            ).wait()


@functools.partial(
    pl.kernel,
    mesh=_mesh,
    compiler_params=_params,
    out_type=jax.ShapeDtypeStruct((B,), jnp.float32),
    scratch_types=[
        pltpu.VMEM((BPW,), jnp.int32),
        pltpu.VMEM((BPW,), jnp.int32),
        pltpu.VMEM((BPW,), jnp.int32),
        pltpu.VMEM((NCH, CH), jnp.int32),
        pltpu.VMEM((NCH, CH), jnp.int32),
        pltpu.VMEM((NCH, CH), jnp.int32),
        pltpu.VMEM((HALF, 128), jnp.float32),
        pltpu.VMEM((HALF, 128), jnp.float32),
        pltpu.VMEM((HALF, 128), jnp.float32),
        pltpu.VMEM((BPW,), jnp.float32),
        pltpu.SemaphoreType.DMA,
    ],
)
def _transe_sc(hidx_hbm, ridx_hbm, tidx_hbm, ent2_hbm, rel2_hbm, out_hbm,
               hs_v, rs_v, ts_v, hk_v, rk_v, tk_v, hD_v, rD_v, tD_v,
               out_v, sem):
    wid = lax.axis_index("s") * NC + lax.axis_index("c")
    base = wid * BPW

    pltpu.sync_copy(hidx_hbm.at[pl.ds(base, BPW)], hs_v)
    pltpu.sync_copy(ridx_hbm.at[pl.ds(base, BPW)], rs_v)
    pltpu.sync_copy(tidx_hbm.at[pl.ds(base, BPW)], ts_v)

    lane = lax.iota(jnp.int32, L)

    for p in range(BPW // HALF):
        def i_body(g, carry):
            col = g * L
            off = p * HALF + col
            for st_v, k_v in ((hs_v, hk_v), (rs_v, rk_v), (ts_v, tk_v)):
                k_v[g >> 3, pl.ds((g & 7) * L, L)] = st_v[pl.ds(off, L)] >> 1
            return carry

        lax.fori_loop(0, NG, i_body, 0)

        cps = []
        for c in range(NCH):
            dst = pl.ds(c * CH, CH)
            cps.append(pltpu.async_copy(ent2_hbm.at[hk_v.at[c]], hD_v.at[dst], sem))
            cps.append(pltpu.async_copy(rel2_hbm.at[rk_v.at[c]], rD_v.at[dst], sem))
            cps.append(pltpu.async_copy(ent2_hbm.at[tk_v.at[c]], tD_v.at[dst], sem))
        for cp in cps:
            cp.wait()

        def c_body(g, carry):
            col = g * L
            off = p * HALF + col
            slots = col + lane
            hc = (hs_v[pl.ds(off, L)] & 1) * D
            rc = (rs_v[pl.ds(off, L)] & 1) * D
            tc = (ts_v[pl.ds(off, L)] & 1) * D

            def d_body(d, acc):
                hv = plsc.load_gather(hD_v, [slots, hc + d])
                rv = plsc.load_gather(rD_v, [slots, rc + d])
                tv = plsc.load_gather(tD_v, [slots, tc + d])
                return acc + jnp.abs(hv + rv - tv)

            acc = lax.fori_loop(0, D, d_body, jnp.zeros((L,), jnp.float32))
            out_v[pl.ds(off, L)] = acc
            return carry

        lax.fori_loop(0, NG, c_body, 0)

    pltpu.sync_copy(out_v, out_hbm.at[pl.ds(base, BPW)])


def kernel(triples, ent, rel):
    tr = triples.astype(jnp.int32)
    e2, r2 = _repack_sc(ent.T, rel.T)
    return _transe_sc(tr[:, 0], tr[:, 1], tr[:, 2], e2, r2)
